# Initial kernel scaffold; baseline (speedup 1.0000x reference)
import jax
import jax.numpy as jnp
from jax.experimental import pallas as pl


def kernel(x, edge_index, W1, b1, bn_gamma, bn_beta, bn_mean, bn_var, W2, b2, cluster_centers, temperature):
    raise NotImplementedError("write your pallas kernel here")


# speedup vs baseline: 18.9354x; 18.9354x over previous
import functools

import jax
import jax.numpy as jnp
from jax import lax
from jax.experimental import pallas as pl
from jax.experimental.pallas import tpu as pltpu
from jax.experimental.pallas import tpu_sc as plsc

N = 10000
E = 320000
D_IN = 128
D_H = 128
D_EMB = 64
K = 16
EPS = 1e-5

NC = 2
NS = 16
NW = NC * NS
EPW = E // NW
C_E = 80
NCHUNK = EPW // C_E
NP = 10240
RPT = NP // NS
ZCH = RPT // C_E
R_TC = 2048
GRID = NP // R_TC

_MESH = plsc.VectorSubcoreMesh(
    core_axis_name="c", subcore_axis_name="s", num_cores=NC, num_subcores=NS)

_HIGH = lax.Precision.HIGHEST


@functools.partial(
    pl.kernel,
    out_type=jax.ShapeDtypeStruct((NC, NP, 16), jnp.float32),
    mesh=_MESH,
    scratch_types=[
        pltpu.VMEM((NCHUNK, C_E), jnp.int32),
        pltpu.VMEM((C_E, 16), jnp.float32),
        pltpu.VMEM_SHARED((NP, 16), jnp.float32),
    ],
)
def _deg_kernel(dst_hbm, out_hbm, dstv, buf, shared):
    cid = lax.axis_index("c")
    sid = lax.axis_index("s")
    wid = sid * NC + cid
    z16 = jnp.zeros((16,), jnp.float32)

    def fz(r, c):
        buf[r, :] = z16
        return c
    lax.fori_loop(0, C_E, fz, 0)

    base = sid * RPT

    def zc(j, c):
        pltpu.sync_copy(buf, shared.at[pl.ds(base + j * C_E, C_E)])
        return c
    lax.fori_loop(0, ZCH, zc, 0)

    o16 = jnp.ones((16,), jnp.float32)

    def fo(r, c):
        buf[r, :] = o16
        return c
    lax.fori_loop(0, C_E, fo, 0)

    pltpu.sync_copy(dst_hbm.at[wid], dstv)
    plsc.subcore_barrier()

    def body(j, c):
        pltpu.sync_copy(buf, shared.at[dstv.at[j]], add=True)
        return c
    lax.fori_loop(0, NCHUNK, body, 0)

    plsc.subcore_barrier()
    pltpu.sync_copy(shared.at[pl.ds(base, RPT)],
                    out_hbm.at[cid, pl.ds(base, RPT)])


def _make_agg(D):
    @functools.partial(
        pl.kernel,
        out_type=jax.ShapeDtypeStruct((NC, NP, D), jnp.float32),
        mesh=_MESH,
        scratch_types=[
            pltpu.VMEM((NCHUNK, C_E), jnp.int32),
            pltpu.VMEM((NCHUNK, C_E), jnp.int32),
            pltpu.VMEM((C_E, D), jnp.float32),
            pltpu.VMEM_SHARED((NP, D), jnp.float32),
            pltpu.SemaphoreType.DMA,
        ],
    )
    def _agg(xs_hbm, src_hbm, dst_hbm, out_hbm, srcv, dstv, rows, shared, sem):
        cid = lax.axis_index("c")
        sid = lax.axis_index("s")
        wid = sid * NC + cid
        z16 = jnp.zeros((16,), jnp.float32)
        pd = D // 16

        def fz(r, c):
            def fz2(q, c2):
                rows[r, pl.ds(q * 16, 16)] = z16
                return c2
            lax.fori_loop(0, pd, fz2, 0)
            return c
        lax.fori_loop(0, C_E, fz, 0)

        base = sid * RPT

        def zc(j, c):
            pltpu.sync_copy(rows, shared.at[pl.ds(base + j * C_E, C_E)])
            return c
        lax.fori_loop(0, ZCH, zc, 0)

        pltpu.sync_copy(src_hbm.at[wid], srcv)
        pltpu.sync_copy(dst_hbm.at[wid], dstv)
        plsc.subcore_barrier()

        def body(j, c):
            pltpu.async_copy(xs_hbm.at[srcv.at[j]], rows, sem).wait()
            pltpu.sync_copy(rows, shared.at[dstv.at[j]], add=True)
            return c
        lax.fori_loop(0, NCHUNK, body, 0)

        plsc.subcore_barrier()
        pltpu.sync_copy(shared.at[pl.ds(base, RPT)],
                        out_hbm.at[cid, pl.ds(base, RPT)])

    return _agg


_agg_h = _make_agg(D_H)


def _tc1_body(x_ref, w_ref, d0_ref, d1_ref, xs_ref, dinv_ref):
    deg = d0_ref[:, 0:1] + d1_ref[:, 0:1] + 1.0
    dinv = lax.rsqrt(deg)
    xw = jnp.dot(x_ref[...], w_ref[...],
                 preferred_element_type=jnp.float32, precision=_HIGH)
    xs_ref[...] = xw * dinv
    dinv_ref[...] = dinv


_tc1 = pl.pallas_call(
    _tc1_body,
    grid=(GRID,),
    in_specs=[
        pl.BlockSpec((R_TC, D_IN), lambda i: (i, 0)),
        pl.BlockSpec((D_IN, D_H), lambda i: (0, 0)),
        pl.BlockSpec((R_TC, 16), lambda i: (i, 0)),
        pl.BlockSpec((R_TC, 16), lambda i: (i, 0)),
    ],
    out_specs=[
        pl.BlockSpec((R_TC, D_H), lambda i: (i, 0)),
        pl.BlockSpec((R_TC, 1), lambda i: (i, 0)),
    ],
    out_shape=[
        jax.ShapeDtypeStruct((NP, D_H), jnp.float32),
        jax.ShapeDtypeStruct((NP, 1), jnp.float32),
    ],
)


def _tc2_body(a0_ref, a1_ref, xs1_ref, dinv_ref, mean_ref, var_ref,
              gamma_ref, beta_ref, b1_ref, w2_ref, xs2_ref):
    dv = dinv_ref[...]
    out1 = (a0_ref[...] + a1_ref[...] + xs1_ref[...]) * dv + b1_ref[...]
    h = (out1 - mean_ref[...]) * lax.rsqrt(var_ref[...] + EPS)
    h = h * gamma_ref[...] + beta_ref[...]
    h = jnp.maximum(h, 0.0)
    xs2 = jnp.dot(h, w2_ref[...],
                  preferred_element_type=jnp.float32,
                  precision=_HIGH) * dv
    xs2_ref[...] = jnp.concatenate(
        [xs2, jnp.zeros((R_TC, D_H - D_EMB), jnp.float32)], axis=1)


_tc2 = pl.pallas_call(
    _tc2_body,
    grid=(GRID,),
    in_specs=[
        pl.BlockSpec((R_TC, D_H), lambda i: (i, 0)),
        pl.BlockSpec((R_TC, D_H), lambda i: (i, 0)),
        pl.BlockSpec((R_TC, D_H), lambda i: (i, 0)),
        pl.BlockSpec((R_TC, 1), lambda i: (i, 0)),
        pl.BlockSpec((1, D_H), lambda i: (0, 0)),
        pl.BlockSpec((1, D_H), lambda i: (0, 0)),
        pl.BlockSpec((1, D_H), lambda i: (0, 0)),
        pl.BlockSpec((1, D_H), lambda i: (0, 0)),
        pl.BlockSpec((1, D_H), lambda i: (0, 0)),
        pl.BlockSpec((D_H, D_EMB), lambda i: (0, 0)),
    ],
    out_specs=[pl.BlockSpec((R_TC, D_H), lambda i: (i, 0))],
    out_shape=[jax.ShapeDtypeStruct((NP, D_H), jnp.float32)],
)


def _tc3_body(a0_ref, a1_ref, xs2_ref, dinv_ref, b2_ref, ct_ref, t_ref,
              emb_ref, soft_ref):
    dv = dinv_ref[...]
    emb = (a0_ref[...] + a1_ref[...] + xs2_ref[...]) * dv + b2_ref[...]
    emb_ref[...] = emb
    logits = jnp.dot(emb, ct_ref[...],
                     preferred_element_type=jnp.float32,
                     precision=_HIGH) / t_ref[0]
    m = jnp.max(logits, axis=1, keepdims=True)
    e = jnp.exp(logits - m)
    soft_ref[...] = e / jnp.sum(e, axis=1, keepdims=True)


_tc3 = pl.pallas_call(
    _tc3_body,
    grid=(GRID,),
    in_specs=[
        pl.BlockSpec((R_TC, D_H), lambda i: (i, 0)),
        pl.BlockSpec((R_TC, D_H), lambda i: (i, 0)),
        pl.BlockSpec((R_TC, D_H), lambda i: (i, 0)),
        pl.BlockSpec((R_TC, 1), lambda i: (i, 0)),
        pl.BlockSpec((1, D_H), lambda i: (0, 0)),
        pl.BlockSpec((D_H, K), lambda i: (0, 0)),
        pl.BlockSpec(memory_space=pltpu.SMEM),
    ],
    out_specs=[
        pl.BlockSpec((R_TC, D_H), lambda i: (i, 0)),
        pl.BlockSpec((R_TC, K), lambda i: (i, 0)),
    ],
    out_shape=[
        jax.ShapeDtypeStruct((NP, D_H), jnp.float32),
        jax.ShapeDtypeStruct((NP, K), jnp.float32),
    ],
)


def kernel(x, edge_index, W1, b1, bn_gamma, bn_beta, bn_mean, bn_var,
           W2, b2, cluster_centers, temperature):
    src_c = edge_index[0].reshape(NW, NCHUNK, C_E)
    dst_c = edge_index[1].reshape(NW, NCHUNK, C_E)
    xp = jnp.pad(x, ((0, NP - N), (0, 0)))

    degp = _deg_kernel(dst_c)
    xs1, dinv = _tc1(xp, W1, degp[0], degp[1])

    a1 = _agg_h(xs1, src_c, dst_c)
    (xs2,) = _tc2(a1[0], a1[1], xs1, dinv,
                  bn_mean[None, :], bn_var[None, :],
                  bn_gamma[None, :], bn_beta[None, :],
                  b1[None, :], W2)

    a2 = _agg_h(xs2, src_c, dst_c)
    b2p = jnp.pad(b2, (0, D_H - D_EMB))
    ctp = jnp.pad(cluster_centers.T, ((0, D_H - D_EMB), (0, 0)))
    emb, soft = _tc3(a2[0], a2[1], xs2, dinv,
                     b2p[None, :], ctp, temperature)
    return emb[:N, :D_EMB], soft[:N]
